# baseline (speedup 1.0000x reference)
import jax
import jax.numpy as jnp
from jax.experimental import pallas as pl

N_CODES = 1024
DIM = 128
HW = 1024
BATCH = 16


def _vq_body(z_ref, e_ref, iota_ref, q_ref, idx_ref, loss_ref):
    b = pl.program_id(0)
    z = z_ref[0].reshape(DIM, HW)
    e = e_ref[...]

    in_norm = jnp.sum(z * z, axis=0, keepdims=True)
    e_norm = jnp.sum(e * e, axis=1, keepdims=True)
    e2 = e + e
    dot2_t = jax.lax.dot_general(e2, z, (((1,), (0,)), ((), ())))
    d = (in_norm + e_norm) - dot2_t

    d_min = jnp.min(d, axis=0, keepdims=True)
    code_iota = iota_ref[...]
    masked = jnp.where(d == d_min, code_iota, float(N_CODES))
    idx_f = jnp.min(masked, axis=0, keepdims=True)

    onehot = (code_iota == idx_f).astype(jnp.bfloat16)
    e_hi = e.astype(jnp.bfloat16)
    e_lo = (e - e_hi.astype(jnp.float32)).astype(jnp.bfloat16)
    dims = (((0,), (0,)), ((), ()))
    q_t = (
        jax.lax.dot_general(e_hi, onehot, dims, preferred_element_type=jnp.float32)
        + jax.lax.dot_general(e_lo, onehot, dims, preferred_element_type=jnp.float32)
    )

    diff = q_t - z
    q_ref[0] = (z + diff).reshape(DIM, 32, 32)
    idx_ref[0] = idx_f

    @pl.when(b == 0)
    def _init():
        loss_ref[...] = jnp.zeros((1, 1), jnp.float32)

    loss_ref[...] += jnp.sum(diff * diff, keepdims=True)


def kernel(z_e, embed_w):
    iota_col = jnp.arange(N_CODES, dtype=jnp.float32).reshape(N_CODES, 1)
    q4, idx_f, loss = pl.pallas_call(
        _vq_body,
        grid=(BATCH,),
        in_specs=[
            pl.BlockSpec((1, DIM, 32, 32), lambda b: (b, 0, 0, 0)),
            pl.BlockSpec((N_CODES, DIM), lambda b: (0, 0)),
            pl.BlockSpec((N_CODES, 1), lambda b: (0, 0)),
        ],
        out_specs=[
            pl.BlockSpec((1, DIM, 32, 32), lambda b: (b, 0, 0, 0)),
            pl.BlockSpec((1, 1, HW), lambda b: (b, 0, 0)),
            pl.BlockSpec((1, 1), lambda b: (0, 0)),
        ],
        out_shape=[
            jax.ShapeDtypeStruct((BATCH, DIM, 32, 32), jnp.float32),
            jax.ShapeDtypeStruct((BATCH, 1, HW), jnp.float32),
            jax.ShapeDtypeStruct((1, 1), jnp.float32),
        ],
    )(z_e, embed_w, iota_col)
    indices = idx_f.reshape(BATCH, HW).astype(jnp.int32)
    n_elems = BATCH * DIM * HW
    commitment = (loss[0, 0] / n_elems) * 0.25
    return (q4, indices, commitment)


# speedup vs baseline: 1.7350x; 1.7350x over previous
import jax
import jax.numpy as jnp
from jax.experimental import pallas as pl

N_CODES = 1024
DIM = 128
HW = 1024
BATCH = 16


def _vq_body(z_ref, e_ref, e2_ref, ecat_ref, iota_ref, q_ref, idx_ref, loss_ref):
    b = pl.program_id(0)
    z = z_ref[0]
    e = e_ref[...]

    in_norm = jnp.sum(z * z, axis=0, keepdims=True)
    e_norm = jnp.sum(e * e, axis=1, keepdims=True)
    dot2_t = jax.lax.dot_general(
        e2_ref[...], z, (((1,), (0,)), ((), ())))
    d = (in_norm + e_norm) - dot2_t

    d_min = jnp.min(d, axis=0, keepdims=True)
    code_iota = iota_ref[...]
    masked = jnp.where(d == d_min, code_iota, float(N_CODES))
    idx_f = jnp.min(masked, axis=0, keepdims=True)

    onehot = (code_iota == idx_f).astype(jnp.bfloat16)
    qq = jax.lax.dot_general(
        ecat_ref[...], onehot, (((0,), (0,)), ((), ())),
        preferred_element_type=jnp.float32)
    q_t = qq[:DIM, :] + qq[DIM:, :]

    diff = q_t - z
    q_ref[0] = z + diff
    idx_ref[0] = idx_f

    @pl.when(b == 0)
    def _init():
        loss_ref[...] = jnp.zeros((1, 1), jnp.float32)

    loss_ref[...] += jnp.sum(diff * diff, keepdims=True)


def kernel(z_e, embed_w):
    z3 = z_e.reshape(BATCH, DIM, HW)
    iota_col = jnp.arange(N_CODES, dtype=jnp.float32).reshape(N_CODES, 1)
    e_hi = embed_w.astype(jnp.bfloat16)
    e_lo = (embed_w - e_hi.astype(jnp.float32)).astype(jnp.bfloat16)
    e_cat = jnp.concatenate([e_hi, e_lo], axis=1)
    e2 = embed_w + embed_w
    q3, idx_f, loss = pl.pallas_call(
        _vq_body,
        grid=(BATCH,),
        in_specs=[
            pl.BlockSpec((1, DIM, HW), lambda b: (b, 0, 0)),
            pl.BlockSpec((N_CODES, DIM), lambda b: (0, 0)),
            pl.BlockSpec((N_CODES, DIM), lambda b: (0, 0)),
            pl.BlockSpec((N_CODES, 2 * DIM), lambda b: (0, 0)),
            pl.BlockSpec((N_CODES, 1), lambda b: (0, 0)),
        ],
        out_specs=[
            pl.BlockSpec((1, DIM, HW), lambda b: (b, 0, 0)),
            pl.BlockSpec((1, 1, HW), lambda b: (b, 0, 0)),
            pl.BlockSpec((1, 1), lambda b: (0, 0)),
        ],
        out_shape=[
            jax.ShapeDtypeStruct((BATCH, DIM, HW), jnp.float32),
            jax.ShapeDtypeStruct((BATCH, 1, HW), jnp.float32),
            jax.ShapeDtypeStruct((1, 1), jnp.float32),
        ],
    )(z3, embed_w, e2, e_cat, iota_col)
    quantized_st = q3.reshape(z_e.shape)
    indices = idx_f.reshape(BATCH, HW).astype(jnp.int32)
    n_elems = BATCH * DIM * HW
    commitment = (loss[0, 0] / n_elems) * 0.25
    return (quantized_st, indices, commitment)
